# baseline (speedup 1.0000x reference)
import functools

import jax
import jax.numpy as jnp
from jax import lax
from jax.experimental import pallas as pl
from jax.experimental.pallas import tpu as pltpu
from jax.experimental.pallas import tpu_sc as plsc

_NC = 2
_NS = 16
_NW = _NC * _NS


def _silu(v):
    return v * jax.nn.sigmoid(v)


def _take16(tab, idx):
    dn = jax.lax.GatherDimensionNumbers(
        offset_dims=(), collapsed_slice_dims=(0,), start_index_map=(0,))
    return jax.lax.gather(
        tab, idx[:, None], dn, slice_sizes=(1,),
        mode=jax.lax.GatherScatterMode.PROMISE_IN_BOUNDS)


def _dot_t(a, b):
    return jax.lax.dot_general(a, b, (((1,), (1,)), ((), ())),
                               preferred_element_type=jnp.float32)


def _router_shared_body(x_ref, wg_ref, ws1_ref, ws2_ref,
                        eid_ref, w_ref, rank_ref, counts_ref, sh_ref,
                        carry_ref):
    t = pl.program_id(0)
    E = wg_ref.shape[0]
    TB = x_ref.shape[0]
    xb = x_ref[...]
    s = jax.nn.sigmoid(_dot_t(xb, wg_ref[...]))
    m = jnp.max(s, axis=1)
    ii = jax.lax.broadcasted_iota(jnp.int32, s.shape, 1)
    eid = jnp.min(jnp.where(s >= m[:, None], ii, E), axis=1)
    eid_ref[...] = eid
    w_ref[...] = m / (m + 1e-20)

    @pl.when(t == 0)
    def _():
        carry_ref[...] = jnp.zeros_like(carry_ref)

    oh = (eid[:, None] == ii).astype(jnp.float32)
    ri = jax.lax.broadcasted_iota(jnp.int32, (TB, TB), 0)
    ci = jax.lax.broadcasted_iota(jnp.int32, (TB, TB), 1)
    tril = (ci < ri).astype(jnp.float32)
    local = jax.lax.dot_general(tril, oh, (((1,), (0,)), ((), ())),
                                preferred_element_type=jnp.float32)
    rank = jnp.sum((local + carry_ref[0, :][None, :]) * oh, axis=1)
    rank_ref[...] = rank.astype(jnp.int32)
    carry_ref[...] = carry_ref[...] + jnp.sum(oh, axis=0)[None, :]
    counts_ref[...] = carry_ref[...]

    sh_ref[...] = _dot_t(_silu(_dot_t(xb, ws1_ref[...])), ws2_ref[...])


def _router_shared(xf, Wg, Ws1, Ws2):
    T, D = xf.shape
    E = Wg.shape[0]
    FS = Ws1.shape[0]
    TB = 256
    nt = T // TB
    return pl.pallas_call(
        _router_shared_body,
        grid=(nt,),
        in_specs=[
            pl.BlockSpec((TB, D), lambda t: (t, 0)),
            pl.BlockSpec((E, D), lambda t: (0, 0)),
            pl.BlockSpec((FS, D), lambda t: (0, 0)),
            pl.BlockSpec((D, FS), lambda t: (0, 0)),
        ],
        out_specs=(
            pl.BlockSpec((TB,), lambda t: (t,)),
            pl.BlockSpec((TB,), lambda t: (t,)),
            pl.BlockSpec((TB,), lambda t: (t,)),
            pl.BlockSpec((1, E), lambda t: (0, 0)),
            pl.BlockSpec((TB, D), lambda t: (t, 0)),
        ),
        out_shape=(jax.ShapeDtypeStruct((T,), jnp.int32),
                   jax.ShapeDtypeStruct((T,), jnp.float32),
                   jax.ShapeDtypeStruct((T,), jnp.int32),
                   jax.ShapeDtypeStruct((1, E), jnp.float32),
                   jax.ShapeDtypeStruct((T, D), jnp.float32)),
        scratch_shapes=[pltpu.VMEM((1, E), jnp.float32)],
        compiler_params=pltpu.CompilerParams(
            dimension_semantics=("arbitrary",)),
    )(xf, Wg, Ws1, Ws2)


def _make_sc_scatter(n_tok, n_slot, n_e, d, chunk):
    assert n_tok % (_NW * chunk) == 0 and chunk % 16 == 0
    n_chunks = n_tok // (_NW * chunk)
    mesh = plsc.VectorSubcoreMesh(core_axis_name="c", subcore_axis_name="s")

    @functools.partial(
        pl.kernel, mesh=mesh,
        out_type=(jax.ShapeDtypeStruct((n_slot, d), jnp.float32),
                  jax.ShapeDtypeStruct((n_slot,), jnp.float32)),
        scratch_types=[
            pltpu.VMEM((chunk,), jnp.int32),
            pltpu.VMEM((chunk,), jnp.int32),
            pltpu.VMEM((chunk,), jnp.float32),
            pltpu.VMEM((n_e,), jnp.int32),
            pltpu.VMEM((chunk,), jnp.int32),
            pltpu.VMEM((chunk, d), jnp.float32),
            pltpu.SemaphoreType.DMA,
        ],
    )
    def k(eid_hbm, rank_hbm, w_hbm, po_hbm, rows_hbm, out_hbm, wslot_hbm,
          eid_v, rank_v, w_v, po_v, idx_v, rows_v, sem):
        wid = lax.axis_index("s") * _NC + lax.axis_index("c")
        base0 = wid * (n_chunks * chunk)
        pltpu.sync_copy(po_hbm, po_v)
        for c in range(n_chunks):
            base = base0 + c * chunk
            pltpu.sync_copy(eid_hbm.at[pl.ds(base, chunk)], eid_v)
            pltpu.sync_copy(rank_hbm.at[pl.ds(base, chunk)], rank_v)
            pltpu.sync_copy(w_hbm.at[pl.ds(base, chunk)], w_v)
            d_rows = pltpu.async_copy(
                rows_hbm.at[pl.ds(base, chunk)], rows_v, sem)
            po_all = po_v[...]
            for j in range(chunk // 16):
                sl = pl.ds(j * 16, 16)
                po16 = _take16(po_all, eid_v[sl])
                idx_v[sl] = po16 + rank_v[sl]
            d_rows.wait()
            d1 = pltpu.async_copy(rows_v, out_hbm.at[idx_v], sem)
            d2 = pltpu.async_copy(w_v, wslot_hbm.at[idx_v], sem)
            d1.wait()
            d2.wait()

    return k


def _make_sc_combine(n_tok, n_slot, n_e, d, chunk):
    assert n_tok % (_NW * chunk) == 0 and chunk % 16 == 0
    n_chunks = n_tok // (_NW * chunk)
    nd16 = d // 16
    mesh = plsc.VectorSubcoreMesh(core_axis_name="c", subcore_axis_name="s")

    @functools.partial(
        pl.kernel, mesh=mesh,
        out_type=jax.ShapeDtypeStruct((n_tok, d), jnp.float32),
        scratch_types=[
            pltpu.VMEM((chunk,), jnp.int32),
            pltpu.VMEM((chunk,), jnp.int32),
            pltpu.VMEM((n_e,), jnp.int32),
            pltpu.VMEM((chunk,), jnp.int32),
            pltpu.VMEM((chunk, d), jnp.float32),
            pltpu.VMEM((chunk, d), jnp.float32),
            pltpu.SemaphoreType.DMA,
            pltpu.SemaphoreType.DMA,
        ],
    )
    def k(eid_hbm, rank_hbm, po_hbm, ypad_hbm, sh_hbm, out_hbm,
          eid_v, rank_v, po_v, idx_v, y2, s2, sem, sem2):
        wid = lax.axis_index("s") * _NC + lax.axis_index("c")
        base0 = wid * (n_chunks * chunk)
        pltpu.sync_copy(po_hbm, po_v)
        for c in range(n_chunks):
            base = base0 + c * chunk
            pltpu.sync_copy(eid_hbm.at[pl.ds(base, chunk)], eid_v)
            pltpu.sync_copy(rank_hbm.at[pl.ds(base, chunk)], rank_v)
            d_sh = pltpu.async_copy(sh_hbm.at[pl.ds(base, chunk)], s2, sem2)
            po_all = po_v[...]
            for j in range(chunk // 16):
                sl = pl.ds(j * 16, 16)
                po16 = _take16(po_all, eid_v[sl])
                idx_v[sl] = po16 + rank_v[sl]
            pltpu.async_copy(ypad_hbm.at[idx_v], y2, sem).wait()
            d_sh.wait()

            def body(r, carry):
                for j in range(nd16):
                    sl = pl.ds(j * 16, 16)
                    y2[r, sl] = y2[r, sl] + s2[r, sl]
                return carry

            plsc.parallel_loop(0, chunk, 1, unroll=1,
                               carry=jnp.int32(0))(body)
            pltpu.sync_copy(y2, out_hbm.at[pl.ds(base, chunk)])

    return k


def _group_body(be_ref, nba_ref, x_ref, wsl_ref, w1_ref, w2_ref, out_ref):
    g = pl.program_id(0)

    @pl.when(g < nba_ref[0])
    def _():
        xb = x_ref[...]
        h = _silu(_dot_t(xb, w1_ref[0]))
        y = _dot_t(h, w2_ref[0])
        out_ref[...] = wsl_ref[0, 0, :][:, None] * y


def _grouped_mlp(x_pad, w_slot3, W1, W2, block_expert, nb_act, blk):
    nslot, D = x_pad.shape
    E, F, _ = W1.shape
    nb = nslot // blk
    grid_spec = pltpu.PrefetchScalarGridSpec(
        num_scalar_prefetch=2,
        grid=(nb,),
        in_specs=[
            pl.BlockSpec((blk, D), lambda g, be, nba: (g, 0)),
            pl.BlockSpec((1, 1, blk), lambda g, be, nba: (g, 0, 0)),
            pl.BlockSpec((1, F, D), lambda g, be, nba: (be[g], 0, 0)),
            pl.BlockSpec((1, D, F), lambda g, be, nba: (be[g], 0, 0)),
        ],
        out_specs=pl.BlockSpec((blk, D), lambda g, be, nba: (g, 0)),
    )
    return pl.pallas_call(
        _group_body,
        grid_spec=grid_spec,
        out_shape=jax.ShapeDtypeStruct((nslot, D), jnp.float32),
        compiler_params=pltpu.CompilerParams(
            dimension_semantics=("arbitrary",)),
    )(block_expert, nb_act, x_pad, w_slot3, W1, W2)


_BLK = 128


def kernel(x, Wg, Ws1, Ws2, W1, W2):
    B, T, D = x.shape
    E, F, _ = W1.shape
    xf = x.reshape(T, D)
    blk = _BLK
    nb = T // blk + E
    nslot = nb * blk

    eid, w, rank, counts_f, shared = _router_shared(xf, Wg, Ws1, Ws2)

    counts = counts_f[0].astype(jnp.int32)
    nblocks_e = (counts + blk - 1) // blk
    cum_blocks = jnp.cumsum(nblocks_e)
    pad_offs = (cum_blocks - nblocks_e) * blk
    nb_act = cum_blocks[E - 1].astype(jnp.int32)
    gi = jnp.arange(nb, dtype=jnp.int32)
    block_expert = jnp.sum(
        (gi[:, None] >= cum_blocks[None, :]).astype(jnp.int32), axis=1)
    block_expert = jnp.minimum(block_expert, E - 1)
    last_e = block_expert[jnp.maximum(nb_act - 1, 0)]
    block_expert = jnp.where(gi < nb_act, block_expert, last_e)

    pad_offs = pad_offs.astype(jnp.int32)
    x_pad, w_slot = _make_sc_scatter(T, nslot, E, D, 64)(
        eid, rank, w, pad_offs, xf)
    y_pad = _grouped_mlp(x_pad, w_slot.reshape(nb, 1, blk), W1, W2,
                         block_expert, nb_act[None], blk)
    out = _make_sc_combine(T, nslot, E, D, 32)(
        eid, rank, pad_offs, y_pad, shared)
    return out.reshape(B, T, D)


# speedup vs baseline: 1.0407x; 1.0407x over previous
import functools

import jax
import jax.numpy as jnp
from jax import lax
from jax.experimental import pallas as pl
from jax.experimental.pallas import tpu as pltpu
from jax.experimental.pallas import tpu_sc as plsc

_NC = 2
_NS = 16
_NW = _NC * _NS


def _silu(v):
    return v * jax.nn.sigmoid(v)


def _dot_t(a, b):
    return jax.lax.dot_general(a, b, (((1,), (1,)), ((), ())),
                               preferred_element_type=jnp.float32)


def _router_shared_body(x_ref, wg_ref, ws1_ref, ws2_ref,
                        eid_ref, w_ref, rank_ref, counts_ref, sh_ref,
                        carry_ref):
    t = pl.program_id(0)
    E = wg_ref.shape[0]
    TB = x_ref.shape[0]
    xb = x_ref[...]
    s = jax.nn.sigmoid(_dot_t(xb, wg_ref[...]))
    m = jnp.max(s, axis=1)
    ii = jax.lax.broadcasted_iota(jnp.int32, s.shape, 1)
    eid = jnp.min(jnp.where(s >= m[:, None], ii, E), axis=1)
    eid_ref[...] = eid
    w_ref[...] = m / (m + 1e-20)

    @pl.when(t == 0)
    def _():
        carry_ref[...] = jnp.zeros_like(carry_ref)

    oh = (eid[:, None] == ii).astype(jnp.float32)
    ri = jax.lax.broadcasted_iota(jnp.int32, (TB, TB), 0)
    ci = jax.lax.broadcasted_iota(jnp.int32, (TB, TB), 1)
    tril = (ci < ri).astype(jnp.float32)
    local = jax.lax.dot_general(tril, oh, (((1,), (0,)), ((), ())),
                                preferred_element_type=jnp.float32)
    rank = jnp.sum((local + carry_ref[0, :][None, :]) * oh, axis=1)
    rank_ref[...] = rank.astype(jnp.int32)
    carry_ref[...] = carry_ref[...] + jnp.sum(oh, axis=0)[None, :]
    counts_ref[...] = carry_ref[...]

    sh_ref[...] = _dot_t(_silu(_dot_t(xb, ws1_ref[...])), ws2_ref[...])


def _router_shared(xf, Wg, Ws1, Ws2):
    T, D = xf.shape
    E = Wg.shape[0]
    FS = Ws1.shape[0]
    TB = 256
    nt = T // TB
    return pl.pallas_call(
        _router_shared_body,
        grid=(nt,),
        in_specs=[
            pl.BlockSpec((TB, D), lambda t: (t, 0)),
            pl.BlockSpec((E, D), lambda t: (0, 0)),
            pl.BlockSpec((FS, D), lambda t: (0, 0)),
            pl.BlockSpec((D, FS), lambda t: (0, 0)),
        ],
        out_specs=(
            pl.BlockSpec((TB,), lambda t: (t,)),
            pl.BlockSpec((TB,), lambda t: (t,)),
            pl.BlockSpec((TB,), lambda t: (t,)),
            pl.BlockSpec((1, E), lambda t: (0, 0)),
            pl.BlockSpec((TB, D), lambda t: (t, 0)),
        ),
        out_shape=(jax.ShapeDtypeStruct((T,), jnp.int32),
                   jax.ShapeDtypeStruct((T,), jnp.float32),
                   jax.ShapeDtypeStruct((T,), jnp.int32),
                   jax.ShapeDtypeStruct((1, E), jnp.float32),
                   jax.ShapeDtypeStruct((T, D), jnp.float32)),
        scratch_shapes=[pltpu.VMEM((1, E), jnp.float32)],
        compiler_params=pltpu.CompilerParams(
            dimension_semantics=("arbitrary",)),
    )(xf, Wg, Ws1, Ws2)


def _make_sc_scatter(n_tok, n_slot, d, chunk):
    assert n_tok % (_NW * chunk) == 0
    n_chunks = n_tok // (_NW * chunk)
    mesh = plsc.VectorSubcoreMesh(core_axis_name="c", subcore_axis_name="s")

    @functools.partial(
        pl.kernel, mesh=mesh,
        out_type=jax.ShapeDtypeStruct((n_slot, d), jnp.float32),
        scratch_types=[
            pltpu.VMEM((chunk,), jnp.int32),
            pltpu.VMEM((chunk, d), jnp.float32),
            pltpu.SemaphoreType.DMA,
        ],
    )
    def k(q_hbm, rows_hbm, out_hbm, idx_v, rows_v, sem):
        wid = lax.axis_index("s") * _NC + lax.axis_index("c")
        base0 = wid * (n_chunks * chunk)
        for c in range(n_chunks):
            base = base0 + c * chunk
            pltpu.sync_copy(q_hbm.at[pl.ds(base, chunk)], idx_v)
            pltpu.sync_copy(rows_hbm.at[pl.ds(base, chunk)], rows_v)
            pltpu.async_copy(rows_v, out_hbm.at[idx_v], sem).wait()

    return k


def _make_sc_combine(n_tok, n_slot, d, chunk):
    assert n_tok % (_NW * chunk) == 0
    n_chunks = n_tok // (_NW * chunk)
    nd16 = d // 16
    mesh = plsc.VectorSubcoreMesh(core_axis_name="c", subcore_axis_name="s")

    @functools.partial(
        pl.kernel, mesh=mesh,
        out_type=jax.ShapeDtypeStruct((n_tok, d), jnp.float32),
        scratch_types=[
            pltpu.VMEM((chunk,), jnp.int32),
            pltpu.VMEM((chunk, d), jnp.float32),
            pltpu.VMEM((chunk, d), jnp.float32),
            pltpu.SemaphoreType.DMA,
        ],
    )
    def k(q_hbm, ypad_hbm, sh_hbm, out_hbm, idx_v, y2, s2, sem):
        wid = lax.axis_index("s") * _NC + lax.axis_index("c")
        base0 = wid * (n_chunks * chunk)
        for c in range(n_chunks):
            base = base0 + c * chunk
            pltpu.sync_copy(q_hbm.at[pl.ds(base, chunk)], idx_v)
            pltpu.async_copy(ypad_hbm.at[idx_v], y2, sem).wait()
            pltpu.sync_copy(sh_hbm.at[pl.ds(base, chunk)], s2)

            def body(r, carry):
                for j in range(nd16):
                    sl = pl.ds(j * 16, 16)
                    y2[r, sl] = y2[r, sl] + s2[r, sl]
                return carry

            plsc.parallel_loop(0, chunk, 1, unroll=1,
                               carry=jnp.int32(0))(body)
            pltpu.sync_copy(y2, out_hbm.at[pl.ds(base, chunk)])

    return k


def _group_body(be_ref, nba_ref, x_ref, wsl_ref, w1_ref, w2_ref, out_ref):
    g = pl.program_id(0)

    @pl.when(g < nba_ref[0])
    def _():
        xb = x_ref[...]
        h = _silu(_dot_t(xb, w1_ref[0]))
        y = _dot_t(h, w2_ref[0])
        out_ref[...] = wsl_ref[0, 0, :][:, None] * y


def _grouped_mlp(x_pad, w_slot3, W1, W2, block_expert, nb_act, blk):
    nslot, D = x_pad.shape
    E, F, _ = W1.shape
    nb = nslot // blk
    grid_spec = pltpu.PrefetchScalarGridSpec(
        num_scalar_prefetch=2,
        grid=(nb,),
        in_specs=[
            pl.BlockSpec((blk, D), lambda g, be, nba: (g, 0)),
            pl.BlockSpec((1, 1, blk), lambda g, be, nba: (g, 0, 0)),
            pl.BlockSpec((1, F, D), lambda g, be, nba: (be[g], 0, 0)),
            pl.BlockSpec((1, D, F), lambda g, be, nba: (be[g], 0, 0)),
        ],
        out_specs=pl.BlockSpec((blk, D), lambda g, be, nba: (g, 0)),
    )
    return pl.pallas_call(
        _group_body,
        grid_spec=grid_spec,
        out_shape=jax.ShapeDtypeStruct((nslot, D), jnp.float32),
        compiler_params=pltpu.CompilerParams(
            dimension_semantics=("arbitrary",)),
    )(block_expert, nb_act, x_pad, w_slot3, W1, W2)


_BLK = 128


def kernel(x, Wg, Ws1, Ws2, W1, W2):
    B, T, D = x.shape
    E, F, _ = W1.shape
    xf = x.reshape(T, D)
    blk = _BLK
    nb = T // blk + E
    nslot = nb * blk

    eid, w, rank, counts_f, shared = _router_shared(xf, Wg, Ws1, Ws2)

    counts = counts_f[0].astype(jnp.int32)
    nblocks_e = (counts + blk - 1) // blk
    cum_blocks = jnp.cumsum(nblocks_e)
    pad_offs = (cum_blocks - nblocks_e) * blk
    nb_act = cum_blocks[E - 1].astype(jnp.int32)
    gi = jnp.arange(nb, dtype=jnp.int32)
    block_expert = jnp.sum(
        (gi[:, None] >= cum_blocks[None, :]).astype(jnp.int32), axis=1)
    block_expert = jnp.minimum(block_expert, E - 1)
    last_e = block_expert[jnp.maximum(nb_act - 1, 0)]
    block_expert = jnp.where(gi < nb_act, block_expert, last_e)

    q = pad_offs[eid] + rank
    w_slot = jnp.zeros((nslot,), jnp.float32).at[q].set(w)

    x_pad = _make_sc_scatter(T, nslot, D, 64)(q, xf)
    y_pad = _grouped_mlp(x_pad, w_slot.reshape(nb, 1, blk), W1, W2,
                         block_expert, nb_act[None], blk)
    out = _make_sc_combine(T, nslot, D, 32)(q, y_pad, shared)
    return out.reshape(B, T, D)


# speedup vs baseline: 1.0495x; 1.0084x over previous
import functools

import jax
import jax.numpy as jnp
from jax import lax
from jax.experimental import pallas as pl
from jax.experimental.pallas import tpu as pltpu
from jax.experimental.pallas import tpu_sc as plsc

_NC = 2
_NS = 16
_NW = _NC * _NS


def _silu(v):
    return v * jax.nn.sigmoid(v)


def _dot_t(a, b):
    return jax.lax.dot_general(a, b, (((1,), (1,)), ((), ())),
                               preferred_element_type=jnp.float32)


def _router_shared_body(x_ref, wg_ref, ws1_ref, ws2_ref,
                        eid_ref, w_ref, rank_ref, counts_ref, sh_ref,
                        carry_ref):
    t = pl.program_id(0)
    E = wg_ref.shape[0]
    TB = x_ref.shape[0]
    xb = x_ref[...]
    s = jax.nn.sigmoid(_dot_t(xb, wg_ref[...]))
    m = jnp.max(s, axis=1)
    ii = jax.lax.broadcasted_iota(jnp.int32, s.shape, 1)
    eid = jnp.min(jnp.where(s >= m[:, None], ii, E), axis=1)
    eid_ref[...] = eid
    w_ref[...] = m / (m + 1e-20)

    @pl.when(t == 0)
    def _():
        carry_ref[...] = jnp.zeros_like(carry_ref)

    oh = (eid[:, None] == ii).astype(jnp.float32)
    ri = jax.lax.broadcasted_iota(jnp.int32, (TB, TB), 0)
    ci = jax.lax.broadcasted_iota(jnp.int32, (TB, TB), 1)
    tril = (ci < ri).astype(jnp.float32)
    local = jax.lax.dot_general(tril, oh, (((1,), (0,)), ((), ())),
                                preferred_element_type=jnp.float32)
    rank = jnp.sum((local + carry_ref[0, :][None, :]) * oh, axis=1)
    rank_ref[...] = rank.astype(jnp.int32)
    carry_ref[...] = carry_ref[...] + jnp.sum(oh, axis=0)[None, :]
    counts_ref[...] = carry_ref[...]

    sh_ref[...] = _dot_t(_silu(_dot_t(xb, ws1_ref[...])), ws2_ref[...])


def _router_shared(xf, Wg, Ws1, Ws2):
    T, D = xf.shape
    E = Wg.shape[0]
    FS = Ws1.shape[0]
    TB = 256
    nt = T // TB
    return pl.pallas_call(
        _router_shared_body,
        grid=(nt,),
        in_specs=[
            pl.BlockSpec((TB, D), lambda t: (t, 0)),
            pl.BlockSpec((E, D), lambda t: (0, 0)),
            pl.BlockSpec((FS, D), lambda t: (0, 0)),
            pl.BlockSpec((D, FS), lambda t: (0, 0)),
        ],
        out_specs=(
            pl.BlockSpec((TB,), lambda t: (t,)),
            pl.BlockSpec((TB,), lambda t: (t,)),
            pl.BlockSpec((TB,), lambda t: (t,)),
            pl.BlockSpec((1, E), lambda t: (0, 0)),
            pl.BlockSpec((TB, D), lambda t: (t, 0)),
        ),
        out_shape=(jax.ShapeDtypeStruct((T,), jnp.int32),
                   jax.ShapeDtypeStruct((T,), jnp.float32),
                   jax.ShapeDtypeStruct((T,), jnp.int32),
                   jax.ShapeDtypeStruct((1, E), jnp.float32),
                   jax.ShapeDtypeStruct((T, D), jnp.float32)),
        scratch_shapes=[pltpu.VMEM((1, E), jnp.float32)],
        compiler_params=pltpu.CompilerParams(
            dimension_semantics=("arbitrary",)),
    )(xf, Wg, Ws1, Ws2)


def _make_sc_scatter(n_tok, n_slot, d, chunk):
    assert n_tok % (_NW * chunk) == 0
    n_chunks = n_tok // (_NW * chunk)
    mesh = plsc.VectorSubcoreMesh(core_axis_name="c", subcore_axis_name="s")

    @functools.partial(
        pl.kernel, mesh=mesh,
        out_type=jax.ShapeDtypeStruct((n_slot, d), jnp.float32),
        scratch_types=[
            pltpu.VMEM((chunk,), jnp.int32),
            pltpu.VMEM((chunk, d), jnp.float32),
            pltpu.SemaphoreType.DMA,
        ],
    )
    def k(q_hbm, rows_hbm, out_hbm, idx_v, rows_v, sem):
        wid = lax.axis_index("s") * _NC + lax.axis_index("c")
        base0 = wid * (n_chunks * chunk)
        for c in range(n_chunks):
            base = base0 + c * chunk
            pltpu.sync_copy(q_hbm.at[pl.ds(base, chunk)], idx_v)
            pltpu.sync_copy(rows_hbm.at[pl.ds(base, chunk)], rows_v)
            pltpu.async_copy(rows_v, out_hbm.at[idx_v], sem).wait()

    return k


def _make_sc_combine(n_tok, n_slot, d, chunk):
    assert n_tok % (_NW * chunk) == 0
    n_chunks = n_tok // (_NW * chunk)
    nd16 = d // 16
    mesh = plsc.VectorSubcoreMesh(core_axis_name="c", subcore_axis_name="s")

    @functools.partial(
        pl.kernel, mesh=mesh,
        out_type=jax.ShapeDtypeStruct((n_tok, d), jnp.float32),
        scratch_types=[
            pltpu.VMEM((chunk,), jnp.int32),
            pltpu.VMEM((chunk, d), jnp.float32),
            pltpu.VMEM((chunk, d), jnp.float32),
            pltpu.SemaphoreType.DMA,
            pltpu.SemaphoreType.DMA,
        ],
    )
    def k(q_hbm, ypad_hbm, sh_hbm, out_hbm, idx_v, y2, s2, sem, sem2):
        wid = lax.axis_index("s") * _NC + lax.axis_index("c")
        base0 = wid * (n_chunks * chunk)
        for c in range(n_chunks):
            base = base0 + c * chunk
            pltpu.sync_copy(q_hbm.at[pl.ds(base, chunk)], idx_v)
            d_sh = pltpu.async_copy(sh_hbm.at[pl.ds(base, chunk)], s2, sem2)
            pltpu.async_copy(ypad_hbm.at[idx_v], y2, sem).wait()
            d_sh.wait()

            def body(r, carry):
                for j in range(nd16):
                    sl = pl.ds(j * 16, 16)
                    y2[r, sl] = y2[r, sl] + s2[r, sl]
                return carry

            plsc.parallel_loop(0, chunk, 1, unroll=1,
                               carry=jnp.int32(0))(body)
            pltpu.sync_copy(y2, out_hbm.at[pl.ds(base, chunk)])

    return k


def _group_body(be_ref, nba_ref, x_ref, wsl_ref, w1_ref, w2_ref, out_ref):
    g = pl.program_id(0)

    @pl.when(g < nba_ref[0])
    def _():
        xb = x_ref[...]
        h = _silu(_dot_t(xb, w1_ref[0]))
        y = _dot_t(h, w2_ref[0])
        out_ref[...] = wsl_ref[0, 0, :][:, None] * y


def _grouped_mlp(x_pad, w_slot3, W1, W2, block_expert, nb_act, blk):
    nslot, D = x_pad.shape
    E, F, _ = W1.shape
    nb = nslot // blk
    grid_spec = pltpu.PrefetchScalarGridSpec(
        num_scalar_prefetch=2,
        grid=(nb,),
        in_specs=[
            pl.BlockSpec((blk, D), lambda g, be, nba: (g, 0)),
            pl.BlockSpec((1, 1, blk), lambda g, be, nba: (g, 0, 0)),
            pl.BlockSpec((1, F, D), lambda g, be, nba: (be[g], 0, 0)),
            pl.BlockSpec((1, D, F), lambda g, be, nba: (be[g], 0, 0)),
        ],
        out_specs=pl.BlockSpec((blk, D), lambda g, be, nba: (g, 0)),
    )
    return pl.pallas_call(
        _group_body,
        grid_spec=grid_spec,
        out_shape=jax.ShapeDtypeStruct((nslot, D), jnp.float32),
        compiler_params=pltpu.CompilerParams(
            dimension_semantics=("arbitrary",)),
    )(block_expert, nb_act, x_pad, w_slot3, W1, W2)


_BLK = 128


def kernel(x, Wg, Ws1, Ws2, W1, W2):
    B, T, D = x.shape
    E, F, _ = W1.shape
    xf = x.reshape(T, D)
    blk = _BLK
    nb = T // blk + E
    nslot = nb * blk

    eid, w, rank, counts_f, shared = _router_shared(xf, Wg, Ws1, Ws2)

    counts = counts_f[0].astype(jnp.int32)
    nblocks_e = (counts + blk - 1) // blk
    cum_blocks = jnp.cumsum(nblocks_e)
    pad_offs = (cum_blocks - nblocks_e) * blk
    nb_act = cum_blocks[E - 1].astype(jnp.int32)
    gi = jnp.arange(nb, dtype=jnp.int32)
    block_expert = jnp.sum(
        (gi[:, None] >= cum_blocks[None, :]).astype(jnp.int32), axis=1)
    block_expert = jnp.minimum(block_expert, E - 1)
    last_e = block_expert[jnp.maximum(nb_act - 1, 0)]
    block_expert = jnp.where(gi < nb_act, block_expert, last_e)

    q = pad_offs[eid] + rank
    w_slot = jnp.zeros((nslot,), jnp.float32).at[q].set(w)

    x_pad = _make_sc_scatter(T, nslot, D, 64)(q, xf)
    y_pad = _grouped_mlp(x_pad, w_slot.reshape(nb, 1, blk), W1, W2,
                         block_expert, nb_act[None], blk)
    out = _make_sc_combine(T, nslot, D, 32)(q, y_pad, shared)
    return out.reshape(B, T, D)
